# baseline (device time: 69036 ns/iter reference)
import jax
import jax.numpy as jnp
from jax import lax
from jax.experimental import pallas as pl
from jax.experimental.pallas import tpu as pltpu

N_GLOBAL = 4096
EPS = 1e-5
BLK = 768
SUB = BLK // 128
NB = 8
NO = 4
HALF = NB // 2


def kernel(x, gamma):
    m, n = x.shape
    assert m == NB * BLK
    g2 = gamma.reshape(1, n)

    def body(x_hbm, g_ref, out_hbm, xv, comm, in_sems, out_sems,
             snd, rcv):
        my_x = lax.axis_index("x")
        my_y = lax.axis_index("y")
        nbr = (my_x, 1 - my_y)

        barrier = pltpu.get_barrier_semaphore()
        pl.semaphore_signal(
            barrier, inc=1, device_id=nbr,
            device_id_type=pl.DeviceIdType.MESH,
        )
        pl.semaphore_wait(barrier, 1)

        def in_copy(i):
            return pltpu.make_async_copy(
                x_hbm.at[pl.ds(i * BLK, BLK), :], xv.at[i], in_sems.at[i]
            )

        def out_copy(j):
            return pltpu.make_async_copy(
                xv.at[j], out_hbm.at[pl.ds(j * BLK, BLK), :],
                out_sems.at[j],
            )

        def exchange(h):
            return pltpu.make_async_remote_copy(
                src_ref=comm.at[h],
                dst_ref=comm.at[2 + h],
                send_sem=snd.at[h],
                recv_sem=rcv.at[h],
                device_id=nbr,
                device_id_type=pl.DeviceIdType.MESH,
            )

        gb = g_ref[...].reshape(1, 1, n)

        for i in range(NB):
            in_copy(i).start()

        for i in range(NB):
            in_copy(i).wait()
        for j in range(NB):
            out_copy(j).start()
        for j in range(NB):
            out_copy(j).wait()

    return pl.pallas_call(
        body,
        out_shape=jax.ShapeDtypeStruct((m, n), x.dtype),
        in_specs=[
            pl.BlockSpec(memory_space=pltpu.MemorySpace.HBM),
            pl.BlockSpec((1, n), lambda: (0, 0)),
        ],
        out_specs=pl.BlockSpec(memory_space=pltpu.MemorySpace.HBM),
        scratch_shapes=[
            pltpu.VMEM((NB, BLK, n), jnp.float32),
            pltpu.VMEM((4, HALF * SUB, 128), jnp.float32),
            pltpu.SemaphoreType.DMA((NB,)),
            pltpu.SemaphoreType.DMA((NB,)),
            pltpu.SemaphoreType.DMA((2,)),
            pltpu.SemaphoreType.DMA((2,)),
        ],
        compiler_params=pltpu.CompilerParams(
            collective_id=0,
            vmem_limit_bytes=100 * 1024 * 1024,
        ),
    )(x, g2)


# device time: 53469 ns/iter; 1.2911x vs baseline; 1.2911x over previous
import jax
import jax.numpy as jnp
from jax import lax
from jax.experimental import pallas as pl
from jax.experimental.pallas import tpu as pltpu

N_GLOBAL = 4096
EPS = 1e-5
BLK = 768
SUB = BLK // 128
NB = 8
HALF = NB // 2


def kernel(x, gamma):
    m, n = x.shape
    assert m == NB * BLK
    g2 = gamma.reshape(1, n)

    def body_a(x_hbm, out_inv, xv, comm, in_sems, snd, rcv):
        my_x = lax.axis_index("x")
        my_y = lax.axis_index("y")
        nbr = (my_x, 1 - my_y)

        barrier = pltpu.get_barrier_semaphore()
        pl.semaphore_signal(
            barrier, inc=1, device_id=nbr,
            device_id_type=pl.DeviceIdType.MESH,
        )

        def in_copy(i):
            return pltpu.make_async_copy(
                x_hbm.at[pl.ds(i * BLK, BLK), :], xv.at[i], in_sems.at[i]
            )

        def exchange(h):
            return pltpu.make_async_remote_copy(
                src_ref=comm.at[h],
                dst_ref=comm.at[2 + h],
                send_sem=snd.at[h],
                recv_sem=rcv.at[h],
                device_id=nbr,
                device_id_type=pl.DeviceIdType.MESH,
            )

        for i in range(NB):
            in_copy(i).start()

        for h in range(2):
            for i in range(h * HALF, (h + 1) * HALF):
                in_copy(i).wait()
                x3 = xv[i].reshape(SUB, 128, n)
                part = jnp.sum(x3 * x3, axis=2)
                comm[h, pl.ds((i % HALF) * SUB, SUB)] = part
            exchange(h).start()

        pl.semaphore_wait(barrier, 1)
        for h in range(2):
            exchange(h).wait_recv()
            total = comm[h] + comm[2 + h]
            out_inv[pl.ds(h * HALF * SUB, HALF * SUB), :] = lax.rsqrt(
                total * (1.0 / N_GLOBAL) + EPS
            )
        exchange(0).wait_send()
        exchange(1).wait_send()

    inv = pl.pallas_call(
        body_a,
        out_shape=jax.ShapeDtypeStruct((NB * SUB, 128), jnp.float32),
        in_specs=[pl.BlockSpec(memory_space=pltpu.MemorySpace.HBM)],
        out_specs=pl.BlockSpec((NB * SUB, 128), lambda: (0, 0)),
        scratch_shapes=[
            pltpu.VMEM((NB, BLK, n), jnp.float32),
            pltpu.VMEM((4, HALF * SUB, 128), jnp.float32),
            pltpu.SemaphoreType.DMA((NB,)),
            pltpu.SemaphoreType.DMA((2,)),
            pltpu.SemaphoreType.DMA((2,)),
        ],
        compiler_params=pltpu.CompilerParams(
            collective_id=0,
            vmem_limit_bytes=100 * 1024 * 1024,
        ),
    )(x)

    def body_b(x_hbm, inv_ref, g_ref, out_hbm, xv, in_sems, out_sems):
        def in_copy(i):
            return pltpu.make_async_copy(
                x_hbm.at[pl.ds(i * BLK, BLK), :], xv.at[i], in_sems.at[i]
            )

        def out_copy(j):
            return pltpu.make_async_copy(
                xv.at[j], out_hbm.at[pl.ds(j * BLK, BLK), :],
                out_sems.at[j],
            )

        for i in range(NB):
            in_copy(i).start()

        gb = g_ref[...].reshape(1, 1, n)
        inv_all = inv_ref[...]
        for i in range(NB):
            in_copy(i).wait()
            inv_i = inv_all[i * SUB:(i + 1) * SUB]
            x3 = xv[i].reshape(SUB, 128, n)
            xv[i] = (x3 * inv_i[:, :, None] * gb).reshape(BLK, n)
            out_copy(i).start()
        for j in range(NB):
            out_copy(j).wait()

    return pl.pallas_call(
        body_b,
        out_shape=jax.ShapeDtypeStruct((m, n), x.dtype),
        in_specs=[
            pl.BlockSpec(memory_space=pltpu.MemorySpace.HBM),
            pl.BlockSpec((NB * SUB, 128), lambda: (0, 0)),
            pl.BlockSpec((1, n), lambda: (0, 0)),
        ],
        out_specs=pl.BlockSpec(memory_space=pltpu.MemorySpace.HBM),
        scratch_shapes=[
            pltpu.VMEM((NB, BLK, n), jnp.float32),
            pltpu.SemaphoreType.DMA((NB,)),
            pltpu.SemaphoreType.DMA((NB,)),
        ],
        compiler_params=pltpu.CompilerParams(
            vmem_limit_bytes=100 * 1024 * 1024,
        ),
    )(x, inv, g2)
